# baseline (device time: 389638 ns/iter reference)
import functools

import jax
from jax import lax
from jax.experimental import pallas as pl
from jax.experimental.pallas import tpu as pltpu


_N_CHUNKS = 4


def kernel(x, pi):
    def body(x_ref, pi_ref, out_ref, send_sem, recv_sem, copy_sem):
        my_x = lax.axis_index("x")
        my_y = lax.axis_index("y")
        my_z = lax.axis_index("z")
        other_z = 1 - my_z
        target_z = pi_ref[my_z]

        barrier_sem = pltpu.get_barrier_semaphore()
        pl.semaphore_signal(
            barrier_sem,
            inc=1,
            device_id=(my_x, my_y, other_z),
            device_id_type=pl.DeviceIdType.MESH,
        )
        pl.semaphore_wait(barrier_sem, 1)

        @pl.when(target_z != my_z)
        def _():
            n_rows = x_ref.shape[1]
            rows_per = n_rows // _N_CHUNKS
            rdmas = []
            for c in range(_N_CHUNKS):
                sl = pl.ds(c * rows_per, rows_per)
                rdma = pltpu.make_async_remote_copy(
                    src_ref=x_ref.at[:, sl, :],
                    dst_ref=out_ref.at[:, sl, :],
                    send_sem=send_sem.at[c],
                    recv_sem=recv_sem.at[c],
                    device_id=(my_x, my_y, target_z),
                    device_id_type=pl.DeviceIdType.MESH,
                )
                rdma.start()
                rdmas.append(rdma)
            for rdma in rdmas:
                rdma.wait()

        @pl.when(target_z == my_z)
        def _():
            copy = pltpu.make_async_copy(x_ref, out_ref, copy_sem)
            copy.start()
            copy.wait()

        @functools.partial(pl.run_scoped, exit_sem=pltpu.SemaphoreType.REGULAR)
        def _(exit_sem):
            pl.semaphore_signal(
                exit_sem,
                inc=1,
                device_id=(my_x, my_y, other_z),
                device_id_type=pl.DeviceIdType.MESH,
            )
            pl.semaphore_wait(exit_sem, 1)

    return pl.pallas_call(
        body,
        out_shape=jax.ShapeDtypeStruct(x.shape, x.dtype),
        in_specs=[
            pl.BlockSpec(memory_space=pl.ANY),
            pl.BlockSpec(memory_space=pltpu.SMEM),
        ],
        out_specs=pl.BlockSpec(memory_space=pl.ANY),
        scratch_shapes=[
            pltpu.SemaphoreType.DMA((_N_CHUNKS,)),
            pltpu.SemaphoreType.DMA((_N_CHUNKS,)),
            pltpu.SemaphoreType.DMA,
        ],
        compiler_params=pltpu.CompilerParams(collective_id=0),
    )(x, pi)
